# Initial kernel scaffold; baseline (speedup 1.0000x reference)
import jax
import jax.numpy as jnp
from jax.experimental import pallas as pl


def kernel(p, x, o, W, bn_gamma, bn_beta):
    raise NotImplementedError("write your pallas kernel here")


# speedup vs baseline: 3.5339x; 3.5339x over previous
import functools

import jax
import jax.numpy as jnp
from jax import lax
from jax.experimental import pallas as pl
from jax.experimental.pallas import tpu as pltpu
from jax.experimental.pallas import tpu_sc as plsc

N = 50000
M = 12500
K = 16
COUT = 32
NPAD = 50176
MPAD = 12544
QB = 128
KC = 512
NCHUNK = NPAD // KC
QD = 784
BGATH = MPAD * K


def _fps_body(px, py, pz, ox, oy, oz, dists):
    dists[...] = jnp.full((392, 128), 1e10, jnp.float32)
    lane = lax.broadcasted_iota(jnp.int32, (1, 128), 1)
    rowi = lax.broadcasted_iota(jnp.int32, (392, 128), 0)
    lanei = lax.broadcasted_iota(jnp.int32, (392, 128), 1)
    flat = rowi * 128 + lanei
    p0x = px[0, 0]
    p0y = py[0, 0]
    p0z = pz[0, 0]

    def body(i, carry):
        lx, ly, lz, bx, by, bz = carry
        d = ((px[...] - lx) ** 2 + (pz[...] - lz) ** 2) + (py[...] - ly) ** 2
        nd = jnp.minimum(dists[...], d)
        dists[...] = nd
        mx = jnp.max(nd)
        pos = jnp.min(jnp.where(nd == mx, flat, jnp.int32(NPAD)))
        r = pos // 128
        c = pos % 128
        sel = lane == c
        nlx = jnp.sum(jnp.where(sel, px[pl.ds(r, 1), :], 0.0))
        nly = jnp.sum(jnp.where(sel, py[pl.ds(r, 1), :], 0.0))
        nlz = jnp.sum(jnp.where(sel, pz[pl.ds(r, 1), :], 0.0))
        here = lane == (i % 128)
        bx = jnp.where(here, nlx, bx)
        by = jnp.where(here, nly, by)
        bz = jnp.where(here, nlz, bz)

        @pl.when(i % 128 == 127)
        def _flush():
            row = i // 128
            ox[pl.ds(row, 1), :] = bx
            oy[pl.ds(row, 1), :] = by
            oz[pl.ds(row, 1), :] = bz

        return nlx, nly, nlz, bx, by, bz

    first = lane == 0
    init = (
        p0x, p0y, p0z,
        jnp.where(first, p0x, 0.0),
        jnp.where(first, p0y, 0.0),
        jnp.where(first, p0z, 0.0),
    )
    lax.fori_loop(1, MPAD, body, init)


def _fps(px, py, pz):
    out = [jax.ShapeDtypeStruct((98, 128), jnp.float32)] * 3
    return pl.pallas_call(
        _fps_body,
        out_shape=out,
        scratch_shapes=[pltpu.VMEM((392, 128), jnp.float32)],
    )(px, py, pz)


def _knn_body(qx, qy, qz, kx, ky, kz, oid):
    lane = lax.broadcasted_iota(jnp.int32, (QB, KC), 1)
    i16 = lax.broadcasted_iota(jnp.int32, (QB, K), 1)
    qxv = qx[...]
    qyv = qy[...]
    qzv = qz[...]

    def chunk(c, carry):
        tv, ti = carry
        kxc = kx[pl.ds(c, 1), :]
        kyc = ky[pl.ds(c, 1), :]
        kzc = kz[pl.ds(c, 1), :]
        d = ((qxv - kxc) ** 2 + (qzv - kzc) ** 2) + (qyv - kyc) ** 2
        gidx = lane + c * KC
        for _ in range(K):
            m = jnp.min(d, axis=1, keepdims=True)
            eq = d == m
            pos = jnp.min(jnp.where(eq, lane, jnp.int32(KC)), axis=1,
                          keepdims=True)
            onehot = lane == pos
            midx = jnp.sum(jnp.where(onehot, gidx, 0), axis=1, keepdims=True)
            d = jnp.where(onehot, jnp.float32(3.4e38), d)
            w = jnp.max(tv, axis=1, keepdims=True)
            weq = tv == w
            wpos = jnp.min(jnp.where(weq, i16, jnp.int32(K)), axis=1,
                           keepdims=True)
            repl = (i16 == wpos) & (m < w)
            tv = jnp.where(repl, m, tv)
            ti = jnp.where(repl, midx, ti)
        return tv, ti

    tv0 = jnp.full((QB, K), 3.4e38, jnp.float32)
    ti0 = jnp.zeros((QB, K), jnp.int32)
    _, ti = lax.fori_loop(0, NCHUNK, chunk, (tv0, ti0))
    oid[...] = ti


def _knn(qcols, krows):
    return pl.pallas_call(
        _knn_body,
        grid=(MPAD // QB,),
        in_specs=[pl.BlockSpec((QB, 1), lambda i: (i, 0))] * 3
        + [pl.BlockSpec((NCHUNK, KC), lambda i: (0, 0))] * 3,
        out_specs=pl.BlockSpec((QB, K), lambda i: (i, 0)),
        out_shape=jax.ShapeDtypeStruct((MPAD, K), jnp.int32),
    )(*qcols, *krows)


def _mm_body(a, w, o):
    av = a[...]
    wv = w[...]
    acc = av[:, 0:1] * wv[0:1, :]
    for c in range(1, 8):
        acc = acc + av[:, c:c + 1] * wv[c:c + 1, :]
    o[...] = acc


def _project(a, w8):
    rows = a.shape[0]
    rb = 7936
    return pl.pallas_call(
        _mm_body,
        grid=(rows // rb,),
        in_specs=[
            pl.BlockSpec((rb, 8), lambda i: (i, 0)),
            pl.BlockSpec((8, COUT), lambda i: (0, 0)),
        ],
        out_specs=pl.BlockSpec((rb, COUT), lambda i: (i, 0)),
        out_shape=jax.ShapeDtypeStruct((rows, COUT), jnp.float32),
    )(a, w8)


def _sc_gather(table, idx):
    info = plsc.get_sparse_core_info()
    nw = info.num_cores * info.num_subcores
    b_per_w = BGATH // nw
    nch = 4
    ch = b_per_w // nch
    mesh = plsc.VectorSubcoreMesh(core_axis_name="c", subcore_axis_name="s")

    @functools.partial(
        pl.kernel,
        mesh=mesh,
        out_type=jax.ShapeDtypeStruct((BGATH, COUT), jnp.float32),
        compiler_params=pltpu.CompilerParams(use_tc_tiling_on_sc=False),
        scratch_types=[
            pltpu.VMEM((ch,), jnp.int32),
            pltpu.VMEM((ch, COUT), jnp.float32),
            pltpu.SemaphoreType.DMA,
        ],
    )
    def k(table_hbm, idx_hbm, out_hbm, idx_v, rows_v, sem):
        wid = lax.axis_index("s") * info.num_cores + lax.axis_index("c")
        base = wid * b_per_w
        for c in range(nch):
            off = base + c * ch
            pltpu.sync_copy(idx_hbm.at[pl.ds(off, ch)], idx_v)
            pltpu.async_copy(table_hbm.at[idx_v], rows_v, sem).wait()
            pltpu.sync_copy(rows_v, out_hbm.at[pl.ds(off, ch)])

    return k(table, idx)


def _stats_body(rows, qp, zmax, zmin, s1, s2):
    i = pl.program_id(0)
    z = rows[...].reshape(QD, K, COUT) - qp[...][:, None, :]
    zmax[...] = jnp.max(z, axis=1)
    zmin[...] = jnp.min(z, axis=1)
    qind = i * QD + lax.broadcasted_iota(jnp.int32, (QD, 1), 0)
    zs = jnp.where((qind < M)[:, :, None], z, 0.0)

    @pl.when(i == 0)
    def _init():
        s1[...] = jnp.zeros((1, COUT), jnp.float32)
        s2[...] = jnp.zeros((1, COUT), jnp.float32)

    s1[...] += jnp.sum(zs, axis=(0, 1)).reshape(1, COUT)
    s2[...] += jnp.sum(zs * zs, axis=(0, 1)).reshape(1, COUT)


def _stats(rows, qproj):
    return pl.pallas_call(
        _stats_body,
        grid=(MPAD // QD,),
        in_specs=[
            pl.BlockSpec((QD * K, COUT), lambda i: (i, 0)),
            pl.BlockSpec((QD, COUT), lambda i: (i, 0)),
        ],
        out_specs=[
            pl.BlockSpec((QD, COUT), lambda i: (i, 0)),
            pl.BlockSpec((QD, COUT), lambda i: (i, 0)),
            pl.BlockSpec((1, COUT), lambda i: (0, 0)),
            pl.BlockSpec((1, COUT), lambda i: (0, 0)),
        ],
        out_shape=[
            jax.ShapeDtypeStruct((MPAD, COUT), jnp.float32),
            jax.ShapeDtypeStruct((MPAD, COUT), jnp.float32),
            jax.ShapeDtypeStruct((1, COUT), jnp.float32),
            jax.ShapeDtypeStruct((1, COUT), jnp.float32),
        ],
    )(rows, qproj)


def _final_body(zmax, zmin, s1, s2, g, b, out):
    cnt = jnp.float32(M * K)
    mean = s1[...] / cnt
    var = s2[...] / cnt - mean * mean
    scale = g[...] * jax.lax.rsqrt(var + 1e-5)
    bias = b[...] - mean * scale
    hi = jnp.maximum(zmax[...] * scale, zmin[...] * scale)
    out[...] = jnp.maximum(hi + bias, 0.0)


def _final(zmax, zmin, s1, s2, g, b):
    return pl.pallas_call(
        _final_body,
        out_shape=jax.ShapeDtypeStruct((MPAD, COUT), jnp.float32),
    )(zmax, zmin, s1, s2, g, b)


def kernel(p, x, o, W, bn_gamma, bn_beta):
    f32 = jnp.float32
    pad = jnp.broadcast_to(p[0], (NPAD - N, 3))
    pp = jnp.concatenate([p, pad], axis=0)
    px = pp[:, 0].reshape(392, 128)
    py = pp[:, 1].reshape(392, 128)
    pz = pp[:, 2].reshape(392, 128)

    npx, npy, npz = _fps(px, py, pz)
    n_p_full = jnp.stack(
        [npx.reshape(MPAD), npy.reshape(MPAD), npz.reshape(MPAD)], axis=1)
    n_p = n_p_full[:M]

    far = jnp.full((NPAD - N, 3), 1e6, f32)
    kk = jnp.concatenate([p, far], axis=0)
    kx = kk[:, 0].reshape(NCHUNK, KC)
    ky = kk[:, 1].reshape(NCHUNK, KC)
    kz = kk[:, 2].reshape(NCHUNK, KC)
    qcols = [n_p_full[:, j].reshape(MPAD, 1) for j in range(3)]
    nbr = _knn(qcols, (kx, ky, kz))

    keyfeat = jnp.concatenate([p, x], axis=1)
    qfeat = jnp.concatenate([n_p_full, jnp.zeros((MPAD, 3), f32)], axis=1)
    a = jnp.concatenate([keyfeat, qfeat], axis=0)
    a = jnp.concatenate([a, jnp.zeros((N + MPAD, 2), f32)], axis=1)
    rows_pad = 63488
    a = jnp.concatenate([a, jnp.zeros((rows_pad - N - MPAD, 8), f32)], axis=0)
    w8 = jnp.concatenate([W, jnp.zeros((2, COUT), f32)], axis=0)
    proj = _project(a, w8)
    table = proj[:N]
    qproj = proj[N:N + MPAD]

    rows = _sc_gather(table, nbr.reshape(BGATH))
    zmax, zmin, s1, s2 = _stats(rows, qproj)
    y = _final(zmax, zmin, s1, s2,
               bn_gamma.reshape(1, COUT), bn_beta.reshape(1, COUT))[:M]

    n_o = jnp.array([M], dtype=jnp.int32)
    return (n_p, y, n_o)


# speedup vs baseline: 4.0173x; 1.1368x over previous
import functools

import jax
import jax.numpy as jnp
from jax import lax
from jax.experimental import pallas as pl
from jax.experimental.pallas import tpu as pltpu
from jax.experimental.pallas import tpu_sc as plsc

N = 50000
M = 12500
K = 16
COUT = 32
NPAD = 50176
MPAD = 12544
QB = 128
KC = 512
NCHUNK = NPAD // KC
QD = 784
BGATH = MPAD * K


def _fps_body(px, py, pz, ox, oy, oz, dists):
    dists[...] = jnp.full((392, 128), 1e10, jnp.float32)
    lane = lax.broadcasted_iota(jnp.int32, (1, 128), 1)
    rowi = lax.broadcasted_iota(jnp.int32, (392, 128), 0)
    lanei = lax.broadcasted_iota(jnp.int32, (392, 128), 1)
    flat = rowi * 128 + lanei
    p0x = px[0, 0]
    p0y = py[0, 0]
    p0z = pz[0, 0]

    def body(i, carry):
        lx, ly, lz, bx, by, bz = carry
        d = ((px[...] - lx) ** 2 + (pz[...] - lz) ** 2) + (py[...] - ly) ** 2
        nd = jnp.minimum(dists[...], d)
        dists[...] = nd
        mx = jnp.max(nd)
        pos = jnp.min(jnp.where(nd == mx, flat, jnp.int32(NPAD)))
        r = pos // 128
        c = pos % 128
        sel = lane == c
        nlx = jnp.sum(jnp.where(sel, px[pl.ds(r, 1), :], 0.0))
        nly = jnp.sum(jnp.where(sel, py[pl.ds(r, 1), :], 0.0))
        nlz = jnp.sum(jnp.where(sel, pz[pl.ds(r, 1), :], 0.0))
        here = lane == (i % 128)
        bx = jnp.where(here, nlx, bx)
        by = jnp.where(here, nly, by)
        bz = jnp.where(here, nlz, bz)

        @pl.when(i % 128 == 127)
        def _flush():
            row = i // 128
            ox[pl.ds(row, 1), :] = bx
            oy[pl.ds(row, 1), :] = by
            oz[pl.ds(row, 1), :] = bz

        return nlx, nly, nlz, bx, by, bz

    first = lane == 0
    init = (
        p0x, p0y, p0z,
        jnp.where(first, p0x, 0.0),
        jnp.where(first, p0y, 0.0),
        jnp.where(first, p0z, 0.0),
    )
    lax.fori_loop(1, MPAD, body, init)


def _fps(px, py, pz):
    out = [jax.ShapeDtypeStruct((98, 128), jnp.float32)] * 3
    return pl.pallas_call(
        _fps_body,
        out_shape=out,
        scratch_shapes=[pltpu.VMEM((392, 128), jnp.float32)],
    )(px, py, pz)


def _knn_body(qx, qy, qz, kx, ky, kz, oid):
    lane = lax.broadcasted_iota(jnp.int32, (QB, KC), 1)
    i16 = lax.broadcasted_iota(jnp.int32, (QB, K), 1)
    qxv = qx[...]
    qyv = qy[...]
    qzv = qz[...]

    def chunk(c, carry):
        tv, ti = carry
        kxc = kx[pl.ds(c, 1), :]
        kyc = ky[pl.ds(c, 1), :]
        kzc = kz[pl.ds(c, 1), :]
        d = ((qxv - kxc) ** 2 + (qzv - kzc) ** 2) + (qyv - kyc) ** 2
        for _ in range(K):
            m = jnp.min(d, axis=1, keepdims=True)
            eq = d == m
            pos = jnp.min(jnp.where(eq, lane, jnp.int32(KC)), axis=1,
                          keepdims=True)
            midx = pos + c * KC
            d = jnp.where(lane == pos, jnp.float32(3.4e38), d)
            w = jnp.max(tv, axis=1, keepdims=True)
            weq = tv == w
            wpos = jnp.min(jnp.where(weq, i16, jnp.int32(K)), axis=1,
                           keepdims=True)
            repl = (i16 == wpos) & (m < w)
            tv = jnp.where(repl, m, tv)
            ti = jnp.where(repl, midx, ti)
        return tv, ti

    tv0 = jnp.full((QB, K), 3.4e38, jnp.float32)
    ti0 = jnp.zeros((QB, K), jnp.int32)
    _, ti = lax.fori_loop(0, NCHUNK, chunk, (tv0, ti0))
    oid[...] = ti


def _knn(qcols, krows):
    return pl.pallas_call(
        _knn_body,
        grid=(MPAD // QB,),
        in_specs=[pl.BlockSpec((QB, 1), lambda i: (i, 0))] * 3
        + [pl.BlockSpec((NCHUNK, KC), lambda i: (0, 0))] * 3,
        out_specs=pl.BlockSpec((QB, K), lambda i: (i, 0)),
        out_shape=jax.ShapeDtypeStruct((MPAD, K), jnp.int32),
    )(*qcols, *krows)


def _mm_body(a, w, o):
    av = a[...]
    wv = w[...]
    acc = av[:, 0:1] * wv[0:1, :]
    for c in range(1, 8):
        acc = acc + av[:, c:c + 1] * wv[c:c + 1, :]
    o[...] = acc


def _project(a, w8):
    rows = a.shape[0]
    rb = 7936
    return pl.pallas_call(
        _mm_body,
        grid=(rows // rb,),
        in_specs=[
            pl.BlockSpec((rb, 8), lambda i: (i, 0)),
            pl.BlockSpec((8, COUT), lambda i: (0, 0)),
        ],
        out_specs=pl.BlockSpec((rb, COUT), lambda i: (i, 0)),
        out_shape=jax.ShapeDtypeStruct((rows, COUT), jnp.float32),
    )(a, w8)


def _sc_gather(table, idx):
    info = plsc.get_sparse_core_info()
    nw = info.num_cores * info.num_subcores
    b_per_w = BGATH // nw
    nch = 4
    ch = b_per_w // nch
    mesh = plsc.VectorSubcoreMesh(core_axis_name="c", subcore_axis_name="s")

    @functools.partial(
        pl.kernel,
        mesh=mesh,
        out_type=jax.ShapeDtypeStruct((BGATH, COUT), jnp.float32),
        compiler_params=pltpu.CompilerParams(use_tc_tiling_on_sc=False),
        scratch_types=[
            pltpu.VMEM((ch,), jnp.int32),
            pltpu.VMEM((ch, COUT), jnp.float32),
            pltpu.SemaphoreType.DMA,
        ],
    )
    def k(table_hbm, idx_hbm, out_hbm, idx_v, rows_v, sem):
        wid = lax.axis_index("s") * info.num_cores + lax.axis_index("c")
        base = wid * b_per_w
        for c in range(nch):
            off = base + c * ch
            pltpu.sync_copy(idx_hbm.at[pl.ds(off, ch)], idx_v)
            pltpu.async_copy(table_hbm.at[idx_v], rows_v, sem).wait()
            pltpu.sync_copy(rows_v, out_hbm.at[pl.ds(off, ch)])

    return k(table, idx)


def _stats_body(rows, qp, zmax, zmin, s1, s2):
    i = pl.program_id(0)
    z = rows[...].reshape(QD, K, COUT) - qp[...][:, None, :]
    zmax[...] = jnp.max(z, axis=1)
    zmin[...] = jnp.min(z, axis=1)
    qind = i * QD + lax.broadcasted_iota(jnp.int32, (QD, 1), 0)
    zs = jnp.where((qind < M)[:, :, None], z, 0.0)

    @pl.when(i == 0)
    def _init():
        s1[...] = jnp.zeros((1, COUT), jnp.float32)
        s2[...] = jnp.zeros((1, COUT), jnp.float32)

    s1[...] += jnp.sum(zs, axis=(0, 1)).reshape(1, COUT)
    s2[...] += jnp.sum(zs * zs, axis=(0, 1)).reshape(1, COUT)


def _stats(rows, qproj):
    return pl.pallas_call(
        _stats_body,
        grid=(MPAD // QD,),
        in_specs=[
            pl.BlockSpec((QD * K, COUT), lambda i: (i, 0)),
            pl.BlockSpec((QD, COUT), lambda i: (i, 0)),
        ],
        out_specs=[
            pl.BlockSpec((QD, COUT), lambda i: (i, 0)),
            pl.BlockSpec((QD, COUT), lambda i: (i, 0)),
            pl.BlockSpec((1, COUT), lambda i: (0, 0)),
            pl.BlockSpec((1, COUT), lambda i: (0, 0)),
        ],
        out_shape=[
            jax.ShapeDtypeStruct((MPAD, COUT), jnp.float32),
            jax.ShapeDtypeStruct((MPAD, COUT), jnp.float32),
            jax.ShapeDtypeStruct((1, COUT), jnp.float32),
            jax.ShapeDtypeStruct((1, COUT), jnp.float32),
        ],
    )(rows, qproj)


def _final_body(zmax, zmin, s1, s2, g, b, out):
    cnt = jnp.float32(M * K)
    mean = s1[...] / cnt
    var = s2[...] / cnt - mean * mean
    scale = g[...] * jax.lax.rsqrt(var + 1e-5)
    bias = b[...] - mean * scale
    hi = jnp.maximum(zmax[...] * scale, zmin[...] * scale)
    out[...] = jnp.maximum(hi + bias, 0.0)


def _final(zmax, zmin, s1, s2, g, b):
    return pl.pallas_call(
        _final_body,
        out_shape=jax.ShapeDtypeStruct((MPAD, COUT), jnp.float32),
    )(zmax, zmin, s1, s2, g, b)


def kernel(p, x, o, W, bn_gamma, bn_beta):
    f32 = jnp.float32
    pad = jnp.broadcast_to(p[0], (NPAD - N, 3))
    pp = jnp.concatenate([p, pad], axis=0)
    px = pp[:, 0].reshape(392, 128)
    py = pp[:, 1].reshape(392, 128)
    pz = pp[:, 2].reshape(392, 128)

    npx, npy, npz = _fps(px, py, pz)
    n_p_full = jnp.stack(
        [npx.reshape(MPAD), npy.reshape(MPAD), npz.reshape(MPAD)], axis=1)
    n_p = n_p_full[:M]

    far = jnp.full((NPAD - N, 3), 1e6, f32)
    kk = jnp.concatenate([p, far], axis=0)
    kx = kk[:, 0].reshape(NCHUNK, KC)
    ky = kk[:, 1].reshape(NCHUNK, KC)
    kz = kk[:, 2].reshape(NCHUNK, KC)
    qcols = [n_p_full[:, j].reshape(MPAD, 1) for j in range(3)]
    nbr = _knn(qcols, (kx, ky, kz))

    keyfeat = jnp.concatenate([p, x], axis=1)
    qfeat = jnp.concatenate([n_p_full, jnp.zeros((MPAD, 3), f32)], axis=1)
    a = jnp.concatenate([keyfeat, qfeat], axis=0)
    a = jnp.concatenate([a, jnp.zeros((N + MPAD, 2), f32)], axis=1)
    rows_pad = 63488
    a = jnp.concatenate([a, jnp.zeros((rows_pad - N - MPAD, 8), f32)], axis=0)
    w8 = jnp.concatenate([W, jnp.zeros((2, COUT), f32)], axis=0)
    proj = _project(a, w8)
    table = proj[:N]
    qproj = proj[N:N + MPAD]

    rows = _sc_gather(table, nbr.reshape(BGATH))
    zmax, zmin, s1, s2 = _stats(rows, qproj)
    y = _final(zmax, zmin, s1, s2,
               bn_gamma.reshape(1, COUT), bn_beta.reshape(1, COUT))[:M]

    n_o = jnp.array([M], dtype=jnp.int32)
    return (n_p, y, n_o)
